# baseline (speedup 1.0000x reference)
import functools

import jax
import jax.numpy as jnp
from jax import lax
from jax.experimental import pallas as pl
from jax.experimental.pallas import tpu as pltpu
from jax.experimental.pallas import tpu_sc as plsc

EPS = 1e-6
INF = 1e10
B = 16384
L = 50
D = 16
U = 100000
I = 1000000
US = 99968
IS = 999936
SENT = 1 << 30
NC = 2
NS = 16
NW = NC * NS
BPW = B // NW
CHUNK = 128
NCHUNK = BPW // CHUNK

_UC_W = 25
_IC_SUB = 61
_BUFN = _IC_SUB * 128


def _k0_stages(w):
    stages = []
    uc0 = jnp.minimum(w * _UC_W, (US // 128) - _UC_W) * 128
    for t in range(2):
        stages.append((0, t, uc0, _UC_W * 128))
    for s in range(5):
        ic0 = jnp.minimum((w * 4 + s) * _IC_SUB, (IS // 128) - _IC_SUB) * 128
        for t in range(2):
            stages.append((1, t, ic0, _BUFN))
    return stages


def _sc_relayout(ue3, ie3, ut_f, it_f, uf, if_, vb0, vb1, rsem, wsem):
    w = lax.axis_index("s") * NC + lax.axis_index("c")
    tabs = (ue3, ie3)
    outs = (uf, if_)
    strides = (US, IS)
    stages = _k0_stages(w)
    bufs = (vb0, vb1)
    reads = []
    writes = []

    def fire_read(k):
        tab, t, l0, n = stages[k]
        l0 = pl.multiple_of(l0, 128)
        reads.append(pltpu.async_copy(
            tabs[tab].at[t].at[:, pl.ds(l0, n)],
            bufs[k % 2].at[:, pl.ds(0, n)], rsem))

    def fire_writes(k):
        tab, t, l0, n = stages[k]
        l0 = pl.multiple_of(l0, 128)
        ws = []
        for s in range(8):
            ws.append(pltpu.async_copy(
                bufs[k % 2].at[s, pl.ds(0, n)],
                outs[tab].at[pl.ds((8 * t + s) * strides[tab] + l0, n)],
                wsem))
        writes.append(ws)

    fire_read(0)
    for k in range(len(stages)):
        reads[k].wait()
        if k + 1 < len(stages):
            fire_read(k + 1)
        if k >= 2:
            for cp in writes[k - 2]:
                cp.wait()
        fire_writes(k)
    for ws in writes[-2:]:
        for cp in ws:
            cp.wait()

    @pl.when(w == NW - 1)
    def _():
        pltpu.sync_copy(ut_f, uf.at[pl.ds(D * US, (U - US) * D)])
        pltpu.sync_copy(it_f, if_.at[pl.ds(D * IS, (I - IS) * D)])


_sc_relayout_call = functools.partial(
    pl.kernel,
    mesh=plsc.VectorSubcoreMesh(core_axis_name="c", subcore_axis_name="s"),
    out_type=[
        jax.ShapeDtypeStruct((D * U,), jnp.float32),
        jax.ShapeDtypeStruct((D * I,), jnp.float32),
    ],
    scratch_types=[
        pltpu.VMEM((8, _BUFN), jnp.float32),
        pltpu.VMEM((8, _BUFN), jnp.float32),
        pltpu.SemaphoreType.DMA,
        pltpu.SemaphoreType.DMA,
    ],
)(_sc_relayout)


def _sc_gather(ue_f, ie_f, a_t, p_t, m_t, b_t, s_t, uids, iids,
               rs_o, a_o, p_o, m_o, b_o, s_o,
               uidx_v, iidx_v, um_v, im_v, uv_v, iv_v,
               a_v, p_v, m_v, b_v, s_v, rs_v, sem, sem2):
    wid = lax.axis_index("s") * NC + lax.axis_index("c")
    row = wid * NCHUNK
    pltpu.sync_copy(uids.at[pl.ds(row, NCHUNK)], uidx_v)
    pltpu.sync_copy(iids.at[pl.ds(row, NCHUNK)], iidx_v)

    for c in range(NCHUNK):
        for g in range(CHUNK // 16):
            sl = pl.ds(g * 16, 16)
            u16 = uidx_v[c, sl]
            i16 = iidx_v[c, sl]
            u16d = u16 * D
            i16d = i16 * D
            um = u16 < US
            im = i16 < IS
            for d in range(D):
                um_v[c * D + d, sl] = jnp.where(um, u16 + (d * US), u16d + d)
                im_v[c * D + d, sl] = jnp.where(im, i16 + (d * IS), i16d + d)
    copies = []
    for c in range(NCHUNK):
        dst = pl.ds(c * CHUNK, CHUNK)
        for d in range(D):
            copies.append(pltpu.async_copy(
                ue_f.at[um_v.at[c * D + d]], uv_v.at[d, dst],
                sem if d % 2 else sem2))
            copies.append(pltpu.async_copy(
                ie_f.at[im_v.at[c * D + d]], iv_v.at[d, dst],
                sem2 if d % 2 else sem))
        copies.append(pltpu.async_copy(a_t.at[iidx_v.at[c]], a_v.at[dst], sem))
        copies.append(pltpu.async_copy(p_t.at[iidx_v.at[c]], p_v.at[dst], sem))
        copies.append(pltpu.async_copy(m_t.at[iidx_v.at[c]], m_v.at[dst], sem))
        copies.append(pltpu.async_copy(b_t.at[iidx_v.at[c]], b_v.at[dst], sem))
        copies.append(pltpu.async_copy(s_t.at[iidx_v.at[c]], s_v.at[dst], sem))
    for cp in copies:
        cp.wait()

    for g in range(BPW // 16):
        sl = pl.ds(g * 16, 16)
        acc = uv_v[0, sl] * iv_v[0, sl]
        for d in range(1, D):
            acc = acc + uv_v[d, sl] * iv_v[d, sl]
        rs_v[sl] = acc
    base = wid * BPW
    pltpu.sync_copy(rs_v, rs_o.at[pl.ds(base, BPW)])
    pltpu.sync_copy(a_v, a_o.at[pl.ds(base, BPW)])
    pltpu.sync_copy(p_v, p_o.at[pl.ds(base, BPW)])
    pltpu.sync_copy(m_v, m_o.at[pl.ds(base, BPW)])
    pltpu.sync_copy(b_v, b_o.at[pl.ds(base, BPW)])
    pltpu.sync_copy(s_v, s_o.at[pl.ds(base, BPW)])


_sc_gather_call = functools.partial(
    pl.kernel,
    mesh=plsc.VectorSubcoreMesh(core_axis_name="c", subcore_axis_name="s"),
    out_type=[jax.ShapeDtypeStruct((B,), jnp.float32)] * 6,
    scratch_types=[
        pltpu.VMEM((NCHUNK, CHUNK), jnp.int32),
        pltpu.VMEM((NCHUNK, CHUNK), jnp.int32),
        pltpu.VMEM((NCHUNK * D, CHUNK), jnp.int32),
        pltpu.VMEM((NCHUNK * D, CHUNK), jnp.int32),
        pltpu.VMEM((D, BPW), jnp.float32),
        pltpu.VMEM((D, BPW), jnp.float32),
        pltpu.VMEM((BPW,), jnp.float32),
        pltpu.VMEM((BPW,), jnp.float32),
        pltpu.VMEM((BPW,), jnp.float32),
        pltpu.VMEM((BPW,), jnp.float32),
        pltpu.VMEM((BPW,), jnp.float32),
        pltpu.VMEM((BPW,), jnp.float32),
        pltpu.SemaphoreType.DMA,
        pltpu.SemaphoreType.DMA,
    ],
    compiler_params=pltpu.CompilerParams(use_tc_tiling_on_sc=False),
)(_sc_gather)


_SQRT_2PI = 2.5066282746310002


_TCR = 16


def _tc_body(ga, rs, a, p, m, bt, sg, t, ht, out):
    galpha = ga[0, 0]
    for j in range(_TCR):
        r = pl.ds(j, 1)
        alpha = jnp.clip(a[r, :] + galpha, 0.0, INF)
        pi = jnp.clip(p[r, :], 0.0, 1.0)
        beta = jnp.clip(bt[r, :], EPS, INF)
        sigma = jnp.clip(sg[r, :], EPS, INF)
        mu = m[r, :]
        ht_j = ht[:, pl.ds(j * 128, 128)]
        dt = jnp.clip(t[r, :] - ht_j, EPS, INF)
        inv_b = 1.0 / beta
        exp_pdf = inv_b * jnp.exp(-dt * inv_b)
        z = (dt - mu) * (1.0 / sigma)
        norm_pdf = jnp.exp(-0.5 * z * z) * (1.0 / (sigma * _SQRT_2PI))
        sum_k = jnp.sum((1.0 - pi) * exp_pdf + pi * norm_pdf,
                        axis=0, keepdims=True)
        out[r, :] = rs[r, :] + alpha * sum_k


def _tc_call(ga, rs, a, p, m, bt, sg, t, ht_t):
    grid = (B // (128 * _TCR),)
    col = lambda i: (i, 0)
    spec1 = pl.BlockSpec((_TCR, 128), col)
    return pl.pallas_call(
        _tc_body,
        grid=grid,
        in_specs=[
            pl.BlockSpec((1, 1), lambda i: (0, 0)),
            spec1, spec1, spec1, spec1, spec1, spec1, spec1,
            pl.BlockSpec((L, _TCR * 128), lambda i: (0, i)),
        ],
        out_specs=spec1,
        out_shape=jax.ShapeDtypeStruct((B // 128, 128), jnp.float32),
    )(ga, rs, a, p, m, bt, sg, t, ht_t)


def kernel(user_emb, item_emb, global_alpha, item_alpha, item_pi, item_mu,
           item_beta, item_sigma, t, history_time, user_ids, item_ids, length):
    uids = user_ids.astype(jnp.int32).reshape(NW * NCHUNK, CHUNK)
    iids = item_ids.astype(jnp.int32).reshape(NW * NCHUNK, CHUNK)
    ue3 = user_emb.T.reshape(2, 8, U)
    ie3 = item_emb.T.reshape(2, 8, I)
    ut_f = user_emb[US:].reshape(-1)
    it_f = item_emb[IS:].reshape(-1)
    uf, if_ = _sc_relayout_call(ue3, ie3, ut_f, it_f)
    rs, a_g, p_g, m_g, b_g, s_g = _sc_gather_call(
        uf, if_, item_alpha, item_pi, item_mu, item_beta,
        item_sigma, uids, iids)
    ga = global_alpha.astype(jnp.float32).reshape(1, 1)
    r2 = lambda x: x.reshape(B // 128, 128)
    out = _tc_call(
        ga, r2(rs), r2(a_g), r2(p_g), r2(m_g), r2(b_g), r2(s_g),
        r2(t), history_time.T)
    return out.reshape(B)


# speedup vs baseline: 1.0790x; 1.0790x over previous
import functools

import jax
import jax.numpy as jnp
from jax import lax
from jax.experimental import pallas as pl
from jax.experimental.pallas import tpu as pltpu
from jax.experimental.pallas import tpu_sc as plsc

EPS = 1e-6
INF = 1e10
B = 16384
L = 50
D = 16
U = 100000
I = 1000000
US = 99968
IS = 999936
SENT = 1 << 30
NC = 2
NS = 16
NW = NC * NS
BPW = B // NW
CHUNK = 128
NCHUNK = BPW // CHUNK

_UC_W = 25
_IC_SUB = 61
_BUFN = _IC_SUB * 128


def _k0_stages(w):
    stages = []
    uc0 = jnp.minimum(w * _UC_W, (US // 128) - _UC_W) * 128
    for t in range(2):
        stages.append((0, t, uc0, _UC_W * 128))
    for s in range(4):
        ic0 = jnp.minimum((w * 4 + s) * _IC_SUB, (IS // 128) - _IC_SUB) * 128
        for t in range(2):
            stages.append((1, t, ic0, _BUFN))
    ic5 = jnp.minimum((w * 4 + 4) * _IC_SUB, (IS // 128) - 4) * 128
    for t in range(2):
        stages.append((1, t, ic5, 4 * 128))
    return stages


def _sc_relayout(ue3, ie3, ut_f, it_f, uf, if_, vb0, vb1, rsem, wsem):
    w = lax.axis_index("s") * NC + lax.axis_index("c")
    tabs = (ue3, ie3)
    outs = (uf, if_)
    strides = (US, IS)
    stages = _k0_stages(w)
    bufs = (vb0, vb1)
    reads = []
    writes = []

    def fire_read(k):
        tab, t, l0, n = stages[k]
        l0 = pl.multiple_of(l0, 128)
        reads.append(pltpu.async_copy(
            tabs[tab].at[t].at[:, pl.ds(l0, n)],
            bufs[k % 2].at[:, pl.ds(0, n)], rsem))

    def fire_writes(k):
        tab, t, l0, n = stages[k]
        l0 = pl.multiple_of(l0, 128)
        ws = []
        for s in range(8):
            ws.append(pltpu.async_copy(
                bufs[k % 2].at[s, pl.ds(0, n)],
                outs[tab].at[pl.ds((8 * t + s) * strides[tab] + l0, n)],
                wsem))
        writes.append(ws)

    fire_read(0)
    for k in range(len(stages)):
        reads[k].wait()
        if k + 1 < len(stages):
            fire_read(k + 1)
        if k >= 2:
            for cp in writes[k - 2]:
                cp.wait()
        fire_writes(k)
    for ws in writes[-2:]:
        for cp in ws:
            cp.wait()

    @pl.when(w == NW - 1)
    def _():
        pltpu.sync_copy(ut_f, uf.at[pl.ds(D * US, (U - US) * D)])
        pltpu.sync_copy(it_f, if_.at[pl.ds(D * IS, (I - IS) * D)])


_sc_relayout_call = functools.partial(
    pl.kernel,
    mesh=plsc.VectorSubcoreMesh(core_axis_name="c", subcore_axis_name="s"),
    out_type=[
        jax.ShapeDtypeStruct((D * U,), jnp.float32),
        jax.ShapeDtypeStruct((D * I,), jnp.float32),
    ],
    scratch_types=[
        pltpu.VMEM((8, _BUFN), jnp.float32),
        pltpu.VMEM((8, _BUFN), jnp.float32),
        pltpu.SemaphoreType.DMA,
        pltpu.SemaphoreType.DMA,
    ],
)(_sc_relayout)


def _sc_gather(ue_f, ie_f, a_t, p_t, m_t, b_t, s_t, uids, iids,
               rs_o, a_o, p_o, m_o, b_o, s_o,
               uidx_v, iidx_v, um_v, im_v, uv_v, iv_v,
               a_v, p_v, m_v, b_v, s_v, rs_v, sem, sem2):
    wid = lax.axis_index("s") * NC + lax.axis_index("c")
    row = wid * NCHUNK
    pltpu.sync_copy(uids.at[pl.ds(row, NCHUNK)], uidx_v)
    pltpu.sync_copy(iids.at[pl.ds(row, NCHUNK)], iidx_v)

    for c in range(NCHUNK):
        for g in range(CHUNK // 16):
            sl = pl.ds(g * 16, 16)
            u16 = uidx_v[c, sl]
            i16 = iidx_v[c, sl]
            u16d = u16 * D
            i16d = i16 * D
            um = u16 < US
            im = i16 < IS
            for d in range(D):
                um_v[c * D + d, sl] = jnp.where(um, u16 + (d * US), u16d + d)
                im_v[c * D + d, sl] = jnp.where(im, i16 + (d * IS), i16d + d)
    copies = []
    for c in range(NCHUNK):
        dst = pl.ds(c * CHUNK, CHUNK)
        for d in range(D):
            copies.append(pltpu.async_copy(
                ue_f.at[um_v.at[c * D + d]], uv_v.at[d, dst],
                sem if d % 2 else sem2))
            copies.append(pltpu.async_copy(
                ie_f.at[im_v.at[c * D + d]], iv_v.at[d, dst],
                sem2 if d % 2 else sem))
        copies.append(pltpu.async_copy(a_t.at[iidx_v.at[c]], a_v.at[dst], sem))
        copies.append(pltpu.async_copy(p_t.at[iidx_v.at[c]], p_v.at[dst], sem))
        copies.append(pltpu.async_copy(m_t.at[iidx_v.at[c]], m_v.at[dst], sem))
        copies.append(pltpu.async_copy(b_t.at[iidx_v.at[c]], b_v.at[dst], sem))
        copies.append(pltpu.async_copy(s_t.at[iidx_v.at[c]], s_v.at[dst], sem))
    for cp in copies:
        cp.wait()

    for g in range(BPW // 16):
        sl = pl.ds(g * 16, 16)
        acc = uv_v[0, sl] * iv_v[0, sl]
        for d in range(1, D):
            acc = acc + uv_v[d, sl] * iv_v[d, sl]
        rs_v[sl] = acc
    base = wid * BPW
    pltpu.sync_copy(rs_v, rs_o.at[pl.ds(base, BPW)])
    pltpu.sync_copy(a_v, a_o.at[pl.ds(base, BPW)])
    pltpu.sync_copy(p_v, p_o.at[pl.ds(base, BPW)])
    pltpu.sync_copy(m_v, m_o.at[pl.ds(base, BPW)])
    pltpu.sync_copy(b_v, b_o.at[pl.ds(base, BPW)])
    pltpu.sync_copy(s_v, s_o.at[pl.ds(base, BPW)])


_sc_gather_call = functools.partial(
    pl.kernel,
    mesh=plsc.VectorSubcoreMesh(core_axis_name="c", subcore_axis_name="s"),
    out_type=[jax.ShapeDtypeStruct((B,), jnp.float32)] * 6,
    scratch_types=[
        pltpu.VMEM((NCHUNK, CHUNK), jnp.int32),
        pltpu.VMEM((NCHUNK, CHUNK), jnp.int32),
        pltpu.VMEM((NCHUNK * D, CHUNK), jnp.int32),
        pltpu.VMEM((NCHUNK * D, CHUNK), jnp.int32),
        pltpu.VMEM((D, BPW), jnp.float32),
        pltpu.VMEM((D, BPW), jnp.float32),
        pltpu.VMEM((BPW,), jnp.float32),
        pltpu.VMEM((BPW,), jnp.float32),
        pltpu.VMEM((BPW,), jnp.float32),
        pltpu.VMEM((BPW,), jnp.float32),
        pltpu.VMEM((BPW,), jnp.float32),
        pltpu.VMEM((BPW,), jnp.float32),
        pltpu.SemaphoreType.DMA,
        pltpu.SemaphoreType.DMA,
    ],
    compiler_params=pltpu.CompilerParams(use_tc_tiling_on_sc=False),
)(_sc_gather)


_SQRT_2PI = 2.5066282746310002


_TCR = 16


def _tc_body(ga, rs, a, p, m, bt, sg, t, ht, out):
    galpha = ga[0, 0]
    for j in range(_TCR):
        r = pl.ds(j, 1)
        alpha = jnp.clip(a[r, :] + galpha, 0.0, INF)
        pi = jnp.clip(p[r, :], 0.0, 1.0)
        beta = jnp.clip(bt[r, :], EPS, INF)
        sigma = jnp.clip(sg[r, :], EPS, INF)
        mu = m[r, :]
        ht_j = ht[:, pl.ds(j * 128, 128)]
        dt = jnp.clip(t[r, :] - ht_j, EPS, INF)
        inv_b = 1.0 / beta
        exp_pdf = inv_b * jnp.exp(-dt * inv_b)
        z = (dt - mu) * (1.0 / sigma)
        norm_pdf = jnp.exp(-0.5 * z * z) * (1.0 / (sigma * _SQRT_2PI))
        sum_k = jnp.sum((1.0 - pi) * exp_pdf + pi * norm_pdf,
                        axis=0, keepdims=True)
        out[r, :] = rs[r, :] + alpha * sum_k


def _tc_call(ga, rs, a, p, m, bt, sg, t, ht_t):
    grid = (B // (128 * _TCR),)
    col = lambda i: (i, 0)
    spec1 = pl.BlockSpec((_TCR, 128), col)
    return pl.pallas_call(
        _tc_body,
        grid=grid,
        in_specs=[
            pl.BlockSpec((1, 1), lambda i: (0, 0)),
            spec1, spec1, spec1, spec1, spec1, spec1, spec1,
            pl.BlockSpec((L, _TCR * 128), lambda i: (0, i)),
        ],
        out_specs=spec1,
        out_shape=jax.ShapeDtypeStruct((B // 128, 128), jnp.float32),
    )(ga, rs, a, p, m, bt, sg, t, ht_t)


def kernel(user_emb, item_emb, global_alpha, item_alpha, item_pi, item_mu,
           item_beta, item_sigma, t, history_time, user_ids, item_ids, length):
    uids = user_ids.astype(jnp.int32).reshape(NW * NCHUNK, CHUNK)
    iids = item_ids.astype(jnp.int32).reshape(NW * NCHUNK, CHUNK)
    ue3 = user_emb.T.reshape(2, 8, U)
    ie3 = item_emb.T.reshape(2, 8, I)
    ut_f = user_emb[US:].reshape(-1)
    it_f = item_emb[IS:].reshape(-1)
    uf, if_ = _sc_relayout_call(ue3, ie3, ut_f, it_f)
    rs, a_g, p_g, m_g, b_g, s_g = _sc_gather_call(
        uf, if_, item_alpha, item_pi, item_mu, item_beta,
        item_sigma, uids, iids)
    ga = global_alpha.astype(jnp.float32).reshape(1, 1)
    r2 = lambda x: x.reshape(B // 128, 128)
    out = _tc_call(
        ga, r2(rs), r2(a_g), r2(p_g), r2(m_g), r2(b_g), r2(s_g),
        r2(t), history_time.T)
    return out.reshape(B)


# speedup vs baseline: 1.0839x; 1.0045x over previous
import functools

import jax
import jax.numpy as jnp
from jax import lax
from jax.experimental import pallas as pl
from jax.experimental.pallas import tpu as pltpu
from jax.experimental.pallas import tpu_sc as plsc

EPS = 1e-6
INF = 1e10
B = 16384
L = 50
D = 16
U = 100000
I = 1000000
US = 99968
IS = 999936
SENT = 1 << 30
NC = 2
NS = 16
NW = NC * NS
BPW = B // NW
CHUNK = 128
NCHUNK = BPW // CHUNK

_UC_W = 25
_IC_SUB = 61
_BUFN = _IC_SUB * 128


def _k0_stages(w):
    stages = []
    uc0 = jnp.minimum(w * _UC_W, (US // 128) - _UC_W) * 128
    for t in range(2):
        stages.append((0, t, uc0, _UC_W * 128))
    for s in range(4):
        ic0 = jnp.minimum((w * 4 + s) * _IC_SUB, (IS // 128) - _IC_SUB) * 128
        for t in range(2):
            stages.append((1, t, ic0, _BUFN))
    ic5 = jnp.minimum((w * 4 + 4) * _IC_SUB, (IS // 128) - 4) * 128
    for t in range(2):
        stages.append((1, t, ic5, 4 * 128))
    return stages


def _sc_relayout(ue3, ie3, ut_f, it_f, uf, if_, vb0, vb1, rsem, wsem):
    w = lax.axis_index("s") * NC + lax.axis_index("c")
    tabs = (ue3, ie3)
    outs = (uf, if_)
    strides = (US, IS)
    stages = _k0_stages(w)
    bufs = (vb0, vb1)
    reads = []
    writes = []

    def fire_read(k):
        tab, t, l0, n = stages[k]
        l0 = pl.multiple_of(l0, 128)
        reads.append(pltpu.async_copy(
            tabs[tab].at[t].at[:, pl.ds(l0, n)],
            bufs[k % 2].at[:, pl.ds(0, n)], rsem))

    def fire_writes(k):
        tab, t, l0, n = stages[k]
        l0 = pl.multiple_of(l0, 128)
        ws = []
        for s in range(8):
            ws.append(pltpu.async_copy(
                bufs[k % 2].at[s, pl.ds(0, n)],
                outs[tab].at[pl.ds((8 * t + s) * strides[tab] + l0, n)],
                wsem))
        writes.append(ws)

    fire_read(0)
    for k in range(len(stages)):
        reads[k].wait()
        if k + 1 < len(stages):
            fire_read(k + 1)
        if k >= 2:
            for cp in writes[k - 2]:
                cp.wait()
        fire_writes(k)
    for ws in writes[-2:]:
        for cp in ws:
            cp.wait()

    @pl.when(w == NW - 1)
    def _():
        pltpu.sync_copy(ut_f, uf.at[pl.ds(D * US, (U - US) * D)])
        pltpu.sync_copy(it_f, if_.at[pl.ds(D * IS, (I - IS) * D)])


_sc_relayout_call = functools.partial(
    pl.kernel,
    mesh=plsc.VectorSubcoreMesh(core_axis_name="c", subcore_axis_name="s"),
    out_type=[
        jax.ShapeDtypeStruct((D * U,), jnp.float32),
        jax.ShapeDtypeStruct((D * I,), jnp.float32),
    ],
    scratch_types=[
        pltpu.VMEM((8, _BUFN), jnp.float32),
        pltpu.VMEM((8, _BUFN), jnp.float32),
        pltpu.SemaphoreType.DMA,
        pltpu.SemaphoreType.DMA,
    ],
)(_sc_relayout)


def _sc_gather(ue_f, ie_f, a_t, p_t, m_t, b_t, s_t, uids, iids,
               rs_o, a_o, p_o, m_o, b_o, s_o,
               uidx_v, iidx_v, um_v, im_v, uv_v, iv_v,
               a_v, p_v, m_v, b_v, s_v, rs_v, sem, sem2):
    wid = lax.axis_index("s") * NC + lax.axis_index("c")
    row = wid * NCHUNK
    pltpu.sync_copy(uids.at[pl.ds(row, NCHUNK)], uidx_v)
    pltpu.sync_copy(iids.at[pl.ds(row, NCHUNK)], iidx_v)

    for c in range(NCHUNK):
        for g in range(CHUNK // 16):
            sl = pl.ds(g * 16, 16)
            u16 = uidx_v[c, sl]
            i16 = iidx_v[c, sl]
            u16d = u16 * D
            i16d = i16 * D
            um = u16 < US
            im = i16 < IS
            for d in range(D):
                um_v[c * D + d, sl] = jnp.where(um, u16 + (d * US), u16d + d)
                im_v[c * D + d, sl] = jnp.where(im, i16 + (d * IS), i16d + d)
    copies = []
    for c in range(NCHUNK):
        dst = pl.ds(c * CHUNK, CHUNK)
        for d in range(D):
            copies.append(pltpu.async_copy(
                ue_f.at[um_v.at[c * D + d]], uv_v.at[d, dst], sem))
            copies.append(pltpu.async_copy(
                ie_f.at[im_v.at[c * D + d]], iv_v.at[d, dst], sem2))
        copies.append(pltpu.async_copy(a_t.at[iidx_v.at[c]], a_v.at[dst], sem))
        copies.append(pltpu.async_copy(p_t.at[iidx_v.at[c]], p_v.at[dst], sem))
        copies.append(pltpu.async_copy(m_t.at[iidx_v.at[c]], m_v.at[dst], sem))
        copies.append(pltpu.async_copy(b_t.at[iidx_v.at[c]], b_v.at[dst], sem))
        copies.append(pltpu.async_copy(s_t.at[iidx_v.at[c]], s_v.at[dst], sem))
    for cp in copies:
        cp.wait()

    for g in range(BPW // 16):
        sl = pl.ds(g * 16, 16)
        acc = uv_v[0, sl] * iv_v[0, sl]
        for d in range(1, D):
            acc = acc + uv_v[d, sl] * iv_v[d, sl]
        rs_v[sl] = acc
    base = wid * BPW
    pltpu.sync_copy(rs_v, rs_o.at[pl.ds(base, BPW)])
    pltpu.sync_copy(a_v, a_o.at[pl.ds(base, BPW)])
    pltpu.sync_copy(p_v, p_o.at[pl.ds(base, BPW)])
    pltpu.sync_copy(m_v, m_o.at[pl.ds(base, BPW)])
    pltpu.sync_copy(b_v, b_o.at[pl.ds(base, BPW)])
    pltpu.sync_copy(s_v, s_o.at[pl.ds(base, BPW)])


_sc_gather_call = functools.partial(
    pl.kernel,
    mesh=plsc.VectorSubcoreMesh(core_axis_name="c", subcore_axis_name="s"),
    out_type=[jax.ShapeDtypeStruct((B,), jnp.float32)] * 6,
    scratch_types=[
        pltpu.VMEM((NCHUNK, CHUNK), jnp.int32),
        pltpu.VMEM((NCHUNK, CHUNK), jnp.int32),
        pltpu.VMEM((NCHUNK * D, CHUNK), jnp.int32),
        pltpu.VMEM((NCHUNK * D, CHUNK), jnp.int32),
        pltpu.VMEM((D, BPW), jnp.float32),
        pltpu.VMEM((D, BPW), jnp.float32),
        pltpu.VMEM((BPW,), jnp.float32),
        pltpu.VMEM((BPW,), jnp.float32),
        pltpu.VMEM((BPW,), jnp.float32),
        pltpu.VMEM((BPW,), jnp.float32),
        pltpu.VMEM((BPW,), jnp.float32),
        pltpu.VMEM((BPW,), jnp.float32),
        pltpu.SemaphoreType.DMA,
        pltpu.SemaphoreType.DMA,
    ],
    compiler_params=pltpu.CompilerParams(use_tc_tiling_on_sc=False),
)(_sc_gather)


_SQRT_2PI = 2.5066282746310002


_TCR = 16


def _tc_body(ga, rs, a, p, m, bt, sg, t, ht, out):
    galpha = ga[0, 0]
    for j in range(_TCR):
        r = pl.ds(j, 1)
        alpha = jnp.clip(a[r, :] + galpha, 0.0, INF)
        pi = jnp.clip(p[r, :], 0.0, 1.0)
        beta = jnp.clip(bt[r, :], EPS, INF)
        sigma = jnp.clip(sg[r, :], EPS, INF)
        mu = m[r, :]
        ht_j = ht[:, pl.ds(j * 128, 128)]
        dt = jnp.clip(t[r, :] - ht_j, EPS, INF)
        inv_b = 1.0 / beta
        exp_pdf = inv_b * jnp.exp(-dt * inv_b)
        z = (dt - mu) * (1.0 / sigma)
        norm_pdf = jnp.exp(-0.5 * z * z) * (1.0 / (sigma * _SQRT_2PI))
        sum_k = jnp.sum((1.0 - pi) * exp_pdf + pi * norm_pdf,
                        axis=0, keepdims=True)
        out[r, :] = rs[r, :] + alpha * sum_k


def _tc_call(ga, rs, a, p, m, bt, sg, t, ht_t):
    grid = (B // (128 * _TCR),)
    col = lambda i: (i, 0)
    spec1 = pl.BlockSpec((_TCR, 128), col)
    return pl.pallas_call(
        _tc_body,
        grid=grid,
        in_specs=[
            pl.BlockSpec((1, 1), lambda i: (0, 0)),
            spec1, spec1, spec1, spec1, spec1, spec1, spec1,
            pl.BlockSpec((L, _TCR * 128), lambda i: (0, i)),
        ],
        out_specs=spec1,
        out_shape=jax.ShapeDtypeStruct((B // 128, 128), jnp.float32),
    )(ga, rs, a, p, m, bt, sg, t, ht_t)


def kernel(user_emb, item_emb, global_alpha, item_alpha, item_pi, item_mu,
           item_beta, item_sigma, t, history_time, user_ids, item_ids, length):
    uids = user_ids.astype(jnp.int32).reshape(NW * NCHUNK, CHUNK)
    iids = item_ids.astype(jnp.int32).reshape(NW * NCHUNK, CHUNK)
    ue3 = user_emb.T.reshape(2, 8, U)
    ie3 = item_emb.T.reshape(2, 8, I)
    ut_f = user_emb[US:].reshape(-1)
    it_f = item_emb[IS:].reshape(-1)
    uf, if_ = _sc_relayout_call(ue3, ie3, ut_f, it_f)
    rs, a_g, p_g, m_g, b_g, s_g = _sc_gather_call(
        uf, if_, item_alpha, item_pi, item_mu, item_beta,
        item_sigma, uids, iids)
    ga = global_alpha.astype(jnp.float32).reshape(1, 1)
    r2 = lambda x: x.reshape(B // 128, 128)
    out = _tc_call(
        ga, r2(rs), r2(a_g), r2(p_g), r2(m_g), r2(b_g), r2(s_g),
        r2(t), history_time.T)
    return out.reshape(B)


# speedup vs baseline: 1.0870x; 1.0029x over previous
import functools

import jax
import jax.numpy as jnp
from jax import lax
from jax.experimental import pallas as pl
from jax.experimental.pallas import tpu as pltpu
from jax.experimental.pallas import tpu_sc as plsc

EPS = 1e-6
INF = 1e10
B = 16384
L = 50
D = 16
U = 100000
I = 1000000
US = 99968
IS = 999936
NC = 2
NS = 16
NW = NC * NS
BPW = B // NW
CHUNK = 128
NCHUNK = BPW // CHUNK

_UC_W = 25
_IC_SUB = 61
_BUFN = _IC_SUB * 128


def _k0_stages(w):
    stages = []
    uc0 = jnp.minimum(w * _UC_W, (US // 128) - _UC_W) * 128
    for t in range(2):
        stages.append((0, t, uc0, _UC_W * 128))
    for s in range(4):
        ic0 = jnp.minimum((w * 4 + s) * _IC_SUB, (IS // 128) - _IC_SUB) * 128
        for t in range(2):
            stages.append((1, t, ic0, _BUFN))
    ic5 = jnp.minimum((w * 4 + 4) * _IC_SUB, (IS // 128) - 4) * 128
    for t in range(2):
        stages.append((1, t, ic5, 4 * 128))
    return stages


def _sc_relayout(ue3, ie3, ut_f, it_f, uf, if_, vb0, vb1, rsem, wsem):
    w = lax.axis_index("s") * NC + lax.axis_index("c")
    tabs = (ue3, ie3)
    outs = (uf, if_)
    strides = (US, IS)
    stages = _k0_stages(w)
    bufs = (vb0, vb1)
    reads = []
    writes = []

    def fire_read(k):
        tab, t, l0, n = stages[k]
        l0 = pl.multiple_of(l0, 128)
        reads.append(pltpu.async_copy(
            tabs[tab].at[t].at[:, pl.ds(l0, n)],
            bufs[k % 2].at[:, pl.ds(0, n)], rsem))

    def fire_writes(k):
        tab, t, l0, n = stages[k]
        l0 = pl.multiple_of(l0, 128)
        ws = []
        for s in range(8):
            ws.append(pltpu.async_copy(
                bufs[k % 2].at[s, pl.ds(0, n)],
                outs[tab].at[pl.ds((8 * t + s) * strides[tab] + l0, n)],
                wsem))
        writes.append(ws)

    fire_read(0)
    for k in range(len(stages)):
        reads[k].wait()
        if k + 1 < len(stages):
            fire_read(k + 1)
        if k >= 2:
            for cp in writes[k - 2]:
                cp.wait()
        fire_writes(k)
    for ws in writes[-2:]:
        for cp in ws:
            cp.wait()

    @pl.when(w == NW - 1)
    def _():
        pltpu.sync_copy(ut_f, uf.at[pl.ds(D * US, (U - US) * D)])
        pltpu.sync_copy(it_f, if_.at[pl.ds(D * IS, (I - IS) * D)])


_sc_relayout_call = functools.partial(
    pl.kernel,
    mesh=plsc.VectorSubcoreMesh(core_axis_name="c", subcore_axis_name="s"),
    out_type=[
        jax.ShapeDtypeStruct((D * U,), jnp.float32),
        jax.ShapeDtypeStruct((D * I,), jnp.float32),
    ],
    scratch_types=[
        pltpu.VMEM((8, _BUFN), jnp.float32),
        pltpu.VMEM((8, _BUFN), jnp.float32),
        pltpu.SemaphoreType.DMA,
        pltpu.SemaphoreType.DMA,
    ],
)(_sc_relayout)


def _sc_gather(ue_f, ie_f, a_t, p_t, m_t, b_t, s_t, uids, iids,
               rs_o, a_o, p_o, m_o, b_o, s_o,
               uidx_v, iidx_v, um_v, im_v, uv_v, iv_v,
               a_v, p_v, m_v, b_v, s_v, rs_v, sem, sem2):
    wid = lax.axis_index("s") * NC + lax.axis_index("c")
    row = wid * NCHUNK
    pltpu.sync_copy(uids.at[pl.ds(row, NCHUNK)], uidx_v)
    pltpu.sync_copy(iids.at[pl.ds(row, NCHUNK)], iidx_v)

    for c in range(NCHUNK):
        for g in range(CHUNK // 16):
            sl = pl.ds(g * 16, 16)
            u16 = uidx_v[c, sl]
            i16 = iidx_v[c, sl]
            u16d = u16 * D
            i16d = i16 * D
            um = u16 < US
            im = i16 < IS
            for d in range(D):
                um_v[c * D + d, sl] = jnp.where(um, u16 + (d * US), u16d + d)
                im_v[c * D + d, sl] = jnp.where(im, i16 + (d * IS), i16d + d)
    copies = []
    for c in range(NCHUNK):
        dst = pl.ds(c * CHUNK, CHUNK)
        for d in range(D):
            copies.append(pltpu.async_copy(
                ue_f.at[um_v.at[c * D + d]], uv_v.at[d, dst], sem))
            copies.append(pltpu.async_copy(
                ie_f.at[im_v.at[c * D + d]], iv_v.at[d, dst], sem2))
        copies.append(pltpu.async_copy(a_t.at[iidx_v.at[c]], a_v.at[dst], sem))
        copies.append(pltpu.async_copy(p_t.at[iidx_v.at[c]], p_v.at[dst], sem))
        copies.append(pltpu.async_copy(m_t.at[iidx_v.at[c]], m_v.at[dst], sem))
        copies.append(pltpu.async_copy(b_t.at[iidx_v.at[c]], b_v.at[dst], sem))
        copies.append(pltpu.async_copy(s_t.at[iidx_v.at[c]], s_v.at[dst], sem))
    for cp in copies:
        cp.wait()

    for g in range(BPW // 16):
        sl = pl.ds(g * 16, 16)
        acc = uv_v[0, sl] * iv_v[0, sl]
        for d in range(1, D):
            acc = acc + uv_v[d, sl] * iv_v[d, sl]
        rs_v[sl] = acc
    base = wid * BPW
    pltpu.sync_copy(rs_v, rs_o.at[pl.ds(base, BPW)])
    pltpu.sync_copy(a_v, a_o.at[pl.ds(base, BPW)])
    pltpu.sync_copy(p_v, p_o.at[pl.ds(base, BPW)])
    pltpu.sync_copy(m_v, m_o.at[pl.ds(base, BPW)])
    pltpu.sync_copy(b_v, b_o.at[pl.ds(base, BPW)])
    pltpu.sync_copy(s_v, s_o.at[pl.ds(base, BPW)])


_sc_gather_call = functools.partial(
    pl.kernel,
    mesh=plsc.VectorSubcoreMesh(core_axis_name="c", subcore_axis_name="s"),
    out_type=[jax.ShapeDtypeStruct((B,), jnp.float32)] * 6,
    scratch_types=[
        pltpu.VMEM((NCHUNK, CHUNK), jnp.int32),
        pltpu.VMEM((NCHUNK, CHUNK), jnp.int32),
        pltpu.VMEM((NCHUNK * D, CHUNK), jnp.int32),
        pltpu.VMEM((NCHUNK * D, CHUNK), jnp.int32),
        pltpu.VMEM((D, BPW), jnp.float32),
        pltpu.VMEM((D, BPW), jnp.float32),
        pltpu.VMEM((BPW,), jnp.float32),
        pltpu.VMEM((BPW,), jnp.float32),
        pltpu.VMEM((BPW,), jnp.float32),
        pltpu.VMEM((BPW,), jnp.float32),
        pltpu.VMEM((BPW,), jnp.float32),
        pltpu.VMEM((BPW,), jnp.float32),
        pltpu.SemaphoreType.DMA,
        pltpu.SemaphoreType.DMA,
    ],
    compiler_params=pltpu.CompilerParams(use_tc_tiling_on_sc=False),
)(_sc_gather)


_SQRT_2PI = 2.5066282746310002


_TCR = 16


def _tc_body(ga, rs, a, p, m, bt, sg, t, ht, out):
    galpha = ga[0, 0]
    for j in range(_TCR):
        r = pl.ds(j, 1)
        alpha = jnp.clip(a[r, :] + galpha, 0.0, INF)
        pi = jnp.clip(p[r, :], 0.0, 1.0)
        beta = jnp.clip(bt[r, :], EPS, INF)
        sigma = jnp.clip(sg[r, :], EPS, INF)
        mu = m[r, :]
        ht_j = ht[:, pl.ds(j * 128, 128)]
        dt = jnp.clip(t[r, :] - ht_j, EPS, INF)
        inv_b = 1.0 / beta
        exp_pdf = inv_b * jnp.exp(-dt * inv_b)
        z = (dt - mu) * (1.0 / sigma)
        norm_pdf = jnp.exp(-0.5 * z * z) * (1.0 / (sigma * _SQRT_2PI))
        sum_k = jnp.sum((1.0 - pi) * exp_pdf + pi * norm_pdf,
                        axis=0, keepdims=True)
        out[r, :] = rs[r, :] + alpha * sum_k


def _tc_call(ga, rs, a, p, m, bt, sg, t, ht_t):
    grid = (B // (128 * _TCR),)
    col = lambda i: (i, 0)
    spec1 = pl.BlockSpec((_TCR, 128), col)
    return pl.pallas_call(
        _tc_body,
        grid=grid,
        in_specs=[
            pl.BlockSpec((1, 1), lambda i: (0, 0)),
            spec1, spec1, spec1, spec1, spec1, spec1, spec1,
            pl.BlockSpec((L, _TCR * 128), lambda i: (0, i)),
        ],
        out_specs=spec1,
        out_shape=jax.ShapeDtypeStruct((B // 128, 128), jnp.float32),
    )(ga, rs, a, p, m, bt, sg, t, ht_t)


def kernel(user_emb, item_emb, global_alpha, item_alpha, item_pi, item_mu,
           item_beta, item_sigma, t, history_time, user_ids, item_ids, length):
    uids = user_ids.astype(jnp.int32).reshape(NW * NCHUNK, CHUNK)
    iids = item_ids.astype(jnp.int32).reshape(NW * NCHUNK, CHUNK)
    ue3 = user_emb.T.reshape(2, 8, U)
    ie3 = item_emb.T.reshape(2, 8, I)
    ut_f = user_emb[US:].reshape(-1)
    it_f = item_emb[IS:].reshape(-1)
    uf, if_ = _sc_relayout_call(ue3, ie3, ut_f, it_f)
    rs, a_g, p_g, m_g, b_g, s_g = _sc_gather_call(
        uf, if_, item_alpha, item_pi, item_mu, item_beta,
        item_sigma, uids, iids)
    ga = global_alpha.astype(jnp.float32).reshape(1, 1)
    r2 = lambda x: x.reshape(B // 128, 128)
    out = _tc_call(
        ga, r2(rs), r2(a_g), r2(p_g), r2(m_g), r2(b_g), r2(s_g),
        r2(t), history_time.T)
    return out.reshape(B)


# speedup vs baseline: 1.0913x; 1.0039x over previous
import functools

import jax
import jax.numpy as jnp
from jax import lax
from jax.experimental import pallas as pl
from jax.experimental.pallas import tpu as pltpu
from jax.experimental.pallas import tpu_sc as plsc

EPS = 1e-6
INF = 1e10
B = 16384
L = 50
D = 16
U = 100000
I = 1000000
US = 99968
IS = 999936
NC = 2
NS = 16
NW = NC * NS
BPW = B // NW
CHUNK = 128
NCHUNK = BPW // CHUNK

_UC_W = 25
_IC_SUB = 61
_BUFN = _IC_SUB * 128


def _k0_stages(w):
    stages = []
    uc0 = jnp.minimum(w * _UC_W, (US // 128) - _UC_W) * 128
    for t in range(2):
        stages.append((0, t, uc0, _UC_W * 128))
    for s in range(4):
        ic0 = jnp.minimum((w * 4 + s) * _IC_SUB, (IS // 128) - _IC_SUB) * 128
        for t in range(2):
            stages.append((1, t, ic0, _BUFN))
    ic5 = jnp.minimum((w * 4 + 4) * _IC_SUB, (IS // 128) - 4) * 128
    for t in range(2):
        stages.append((1, t, ic5, 4 * 128))
    return stages


def _sc_relayout(ue3, ie3, ut_f, it_f, uf, if_, vb0, vb1, rsem, wsem):
    w = lax.axis_index("s") * NC + lax.axis_index("c")
    tabs = (ue3, ie3)
    outs = (uf, if_)
    strides = (US, IS)
    stages = _k0_stages(w)
    bufs = (vb0, vb1)
    reads = []
    writes = []

    def fire_read(k):
        tab, t, l0, n = stages[k]
        l0 = pl.multiple_of(l0, 128)
        reads.append(pltpu.async_copy(
            tabs[tab].at[t].at[:, pl.ds(l0, n)],
            bufs[k % 2].at[:, pl.ds(0, n)], rsem))

    def fire_writes(k):
        tab, t, l0, n = stages[k]
        l0 = pl.multiple_of(l0, 128)
        ws = []
        for s in range(8):
            ws.append(pltpu.async_copy(
                bufs[k % 2].at[s, pl.ds(0, n)],
                outs[tab].at[pl.ds((8 * t + s) * strides[tab] + l0, n)],
                wsem))
        writes.append(ws)

    fire_read(0)
    for k in range(len(stages)):
        reads[k].wait()
        if k + 1 < len(stages):
            fire_read(k + 1)
        if k >= 2:
            for cp in writes[k - 2]:
                cp.wait()
        fire_writes(k)
    for ws in writes[-2:]:
        for cp in ws:
            cp.wait()

    @pl.when(w == NW - 1)
    def _():
        pltpu.sync_copy(ut_f, uf.at[pl.ds(D * US, (U - US) * D)])
        pltpu.sync_copy(it_f, if_.at[pl.ds(D * IS, (I - IS) * D)])


_sc_relayout_call = functools.partial(
    pl.kernel,
    mesh=plsc.VectorSubcoreMesh(core_axis_name="c", subcore_axis_name="s"),
    out_type=[
        jax.ShapeDtypeStruct((D * U,), jnp.float32),
        jax.ShapeDtypeStruct((D * I,), jnp.float32),
    ],
    scratch_types=[
        pltpu.VMEM((8, _BUFN), jnp.float32),
        pltpu.VMEM((8, _BUFN), jnp.float32),
        pltpu.SemaphoreType.DMA,
        pltpu.SemaphoreType.DMA,
    ],
)(_sc_relayout)


def _sc_gather(ue_f, ie_f, a_t, p_t, m_t, b_t, s_t, uids, iids,
               rs_o, a_o, p_o, m_o, b_o, s_o,
               uidx_v, iidx_v, um_v, im_v, uv_v, iv_v,
               a_v, p_v, m_v, b_v, s_v, rs_v, sem, sem2):
    wid = lax.axis_index("s") * NC + lax.axis_index("c")
    row = wid * NCHUNK
    pltpu.sync_copy(uids.at[pl.ds(row, NCHUNK)], uidx_v)
    pltpu.sync_copy(iids.at[pl.ds(row, NCHUNK)], iidx_v)

    copies = []
    for c in range(NCHUNK):
        for g in range(CHUNK // 16):
            sl = pl.ds(g * 16, 16)
            u16 = uidx_v[c, sl]
            i16 = iidx_v[c, sl]
            u16d = u16 * D
            i16d = i16 * D
            um = u16 < US
            im = i16 < IS
            for d in range(D):
                um_v[c * D + d, sl] = jnp.where(um, u16 + (d * US), u16d + d)
                im_v[c * D + d, sl] = jnp.where(im, i16 + (d * IS), i16d + d)
        dst = pl.ds(c * CHUNK, CHUNK)
        for d in range(D):
            copies.append(pltpu.async_copy(
                ue_f.at[um_v.at[c * D + d]], uv_v.at[d, dst], sem))
            copies.append(pltpu.async_copy(
                ie_f.at[im_v.at[c * D + d]], iv_v.at[d, dst], sem2))
        copies.append(pltpu.async_copy(a_t.at[iidx_v.at[c]], a_v.at[dst], sem))
        copies.append(pltpu.async_copy(p_t.at[iidx_v.at[c]], p_v.at[dst], sem))
        copies.append(pltpu.async_copy(m_t.at[iidx_v.at[c]], m_v.at[dst], sem))
        copies.append(pltpu.async_copy(b_t.at[iidx_v.at[c]], b_v.at[dst], sem))
        copies.append(pltpu.async_copy(s_t.at[iidx_v.at[c]], s_v.at[dst], sem))
    for cp in copies:
        cp.wait()

    for g in range(BPW // 16):
        sl = pl.ds(g * 16, 16)
        acc = uv_v[0, sl] * iv_v[0, sl]
        for d in range(1, D):
            acc = acc + uv_v[d, sl] * iv_v[d, sl]
        rs_v[sl] = acc
    base = wid * BPW
    pltpu.sync_copy(rs_v, rs_o.at[pl.ds(base, BPW)])
    pltpu.sync_copy(a_v, a_o.at[pl.ds(base, BPW)])
    pltpu.sync_copy(p_v, p_o.at[pl.ds(base, BPW)])
    pltpu.sync_copy(m_v, m_o.at[pl.ds(base, BPW)])
    pltpu.sync_copy(b_v, b_o.at[pl.ds(base, BPW)])
    pltpu.sync_copy(s_v, s_o.at[pl.ds(base, BPW)])


_sc_gather_call = functools.partial(
    pl.kernel,
    mesh=plsc.VectorSubcoreMesh(core_axis_name="c", subcore_axis_name="s"),
    out_type=[jax.ShapeDtypeStruct((B,), jnp.float32)] * 6,
    scratch_types=[
        pltpu.VMEM((NCHUNK, CHUNK), jnp.int32),
        pltpu.VMEM((NCHUNK, CHUNK), jnp.int32),
        pltpu.VMEM((NCHUNK * D, CHUNK), jnp.int32),
        pltpu.VMEM((NCHUNK * D, CHUNK), jnp.int32),
        pltpu.VMEM((D, BPW), jnp.float32),
        pltpu.VMEM((D, BPW), jnp.float32),
        pltpu.VMEM((BPW,), jnp.float32),
        pltpu.VMEM((BPW,), jnp.float32),
        pltpu.VMEM((BPW,), jnp.float32),
        pltpu.VMEM((BPW,), jnp.float32),
        pltpu.VMEM((BPW,), jnp.float32),
        pltpu.VMEM((BPW,), jnp.float32),
        pltpu.SemaphoreType.DMA,
        pltpu.SemaphoreType.DMA,
    ],
    compiler_params=pltpu.CompilerParams(use_tc_tiling_on_sc=False),
)(_sc_gather)


_SQRT_2PI = 2.5066282746310002


_TCR = 16


def _tc_body(ga, rs, a, p, m, bt, sg, t, ht, out):
    galpha = ga[0, 0]
    for j in range(_TCR):
        r = pl.ds(j, 1)
        alpha = jnp.clip(a[r, :] + galpha, 0.0, INF)
        pi = jnp.clip(p[r, :], 0.0, 1.0)
        beta = jnp.clip(bt[r, :], EPS, INF)
        sigma = jnp.clip(sg[r, :], EPS, INF)
        mu = m[r, :]
        ht_j = ht[:, pl.ds(j * 128, 128)]
        dt = jnp.clip(t[r, :] - ht_j, EPS, INF)
        inv_b = 1.0 / beta
        exp_pdf = inv_b * jnp.exp(-dt * inv_b)
        z = (dt - mu) * (1.0 / sigma)
        norm_pdf = jnp.exp(-0.5 * z * z) * (1.0 / (sigma * _SQRT_2PI))
        sum_k = jnp.sum((1.0 - pi) * exp_pdf + pi * norm_pdf,
                        axis=0, keepdims=True)
        out[r, :] = rs[r, :] + alpha * sum_k


def _tc_call(ga, rs, a, p, m, bt, sg, t, ht_t):
    grid = (B // (128 * _TCR),)
    col = lambda i: (i, 0)
    spec1 = pl.BlockSpec((_TCR, 128), col)
    return pl.pallas_call(
        _tc_body,
        grid=grid,
        in_specs=[
            pl.BlockSpec((1, 1), lambda i: (0, 0)),
            spec1, spec1, spec1, spec1, spec1, spec1, spec1,
            pl.BlockSpec((L, _TCR * 128), lambda i: (0, i)),
        ],
        out_specs=spec1,
        out_shape=jax.ShapeDtypeStruct((B // 128, 128), jnp.float32),
    )(ga, rs, a, p, m, bt, sg, t, ht_t)


def kernel(user_emb, item_emb, global_alpha, item_alpha, item_pi, item_mu,
           item_beta, item_sigma, t, history_time, user_ids, item_ids, length):
    uids = user_ids.astype(jnp.int32).reshape(NW * NCHUNK, CHUNK)
    iids = item_ids.astype(jnp.int32).reshape(NW * NCHUNK, CHUNK)
    ue3 = user_emb.T.reshape(2, 8, U)
    ie3 = item_emb.T.reshape(2, 8, I)
    ut_f = user_emb[US:].reshape(-1)
    it_f = item_emb[IS:].reshape(-1)
    uf, if_ = _sc_relayout_call(ue3, ie3, ut_f, it_f)
    rs, a_g, p_g, m_g, b_g, s_g = _sc_gather_call(
        uf, if_, item_alpha, item_pi, item_mu, item_beta,
        item_sigma, uids, iids)
    ga = global_alpha.astype(jnp.float32).reshape(1, 1)
    r2 = lambda x: x.reshape(B // 128, 128)
    out = _tc_call(
        ga, r2(rs), r2(a_g), r2(p_g), r2(m_g), r2(b_g), r2(s_g),
        r2(t), history_time.T)
    return out.reshape(B)
